# baseline (device time: 434586 ns/iter reference)
import jax
import jax.numpy as jnp
from jax import lax
from jax.experimental import pallas as pl
from jax.experimental.pallas import tpu as pltpu

N_DEV = 4
B = 2
S_LOC = 512
S = 2048
D = 1024
H_LOC = 8
DH = 128
SCALE = 0.08838834764831843


def _neighbor_barrier(left, right):
    barrier = pltpu.get_barrier_semaphore()
    for nbr in (left, right):
        pl.semaphore_signal(
            barrier, inc=1, device_id=(nbr,),
            device_id_type=pl.DeviceIdType.MESH,
        )
    pl.semaphore_wait(barrier, 2)


def _ag_body(x_ref, out_ref, send_sems, recv_sems):
    my = lax.axis_index("i")
    left = lax.rem(my + N_DEV - 1, N_DEV)
    right = lax.rem(my + 1, N_DEV)

    _neighbor_barrier(left, right)

    out_ref[:, pl.ds(my * S_LOC, S_LOC), :] = x_ref[...].astype(jnp.bfloat16)

    for h in range(N_DEV - 1):
        origin = lax.rem(my + N_DEV - h, N_DEV)
        sl = pl.ds(origin * S_LOC, S_LOC)
        rdma = pltpu.make_async_remote_copy(
            src_ref=out_ref.at[:, sl, :],
            dst_ref=out_ref.at[:, sl, :],
            send_sem=send_sems.at[h],
            recv_sem=recv_sems.at[h],
            device_id=(right,),
            device_id_type=pl.DeviceIdType.MESH,
        )
        rdma.start()
        rdma.wait()


def _all_gather(x):
    return pl.pallas_call(
        _ag_body,
        out_shape=jax.ShapeDtypeStruct((B, S, D), jnp.bfloat16),
        in_specs=[pl.BlockSpec(memory_space=pltpu.VMEM)],
        out_specs=pl.BlockSpec(memory_space=pltpu.VMEM),
        scratch_shapes=[
            pltpu.SemaphoreType.DMA((N_DEV - 1,)),
            pltpu.SemaphoreType.DMA((N_DEV - 1,)),
        ],
        compiler_params=pltpu.CompilerParams(collective_id=0),
    )(x)


def _rope_tables():
    pos = lax.broadcasted_iota(jnp.int32, (S, DH), 0).astype(jnp.float32)
    d = lax.broadcasted_iota(jnp.int32, (S, DH), 1)
    half = (d // 2).astype(jnp.float32)
    inv = jnp.exp(-jnp.log(10000.0) * (half * (2.0 / DH)))
    ang = pos * inv
    return jnp.cos(ang), jnp.sin(ang), (d % 2) == 0


def _rope(t, cos, sin, even):
    l = jnp.concatenate([t[:, 1:], t[:, :1]], axis=1)
    r = jnp.concatenate([t[:, -1:], t[:, :-1]], axis=1)
    tr = jnp.where(even, -l, r)
    return t * cos + tr * sin


def _attn_body(x_ref, wq_ref, wk_ref, wv_ref, wo_ref, out_ref):
    h = pl.program_id(1)
    f32 = jnp.float32
    bf16 = jnp.bfloat16
    cos, sin, even = _rope_tables()
    xb = x_ref[0]
    wq = wq_ref[...].astype(bf16)
    wk = wk_ref[...].astype(bf16)
    wv = wv_ref[...].astype(bf16)
    wo = wo_ref[...].astype(bf16)
    q = _rope(jnp.dot(xb, wq, preferred_element_type=f32), cos, sin, even)
    k = _rope(jnp.dot(xb, wk, preferred_element_type=f32), cos, sin, even)
    qb16 = (q * SCALE).astype(bf16)
    kb16 = k.astype(bf16)
    v = jnp.dot(xb, wv, preferred_element_type=f32).astype(bf16)
    v_aug = jnp.concatenate([v, jnp.ones((S, 8), bf16)], axis=1)
    for qb in range(S // S_LOC):
        qs = qb16[qb * S_LOC:(qb + 1) * S_LOC]
        s = lax.dot_general(
            qs, kb16, (((1,), (1,)), ((), ())),
            preferred_element_type=f32,
        )
        p = jnp.exp(s).astype(bf16)
        ctx_aug = jnp.dot(p, v_aug, preferred_element_type=f32)
        denom = ctx_aug[:, DH:DH + 1]
        ctx = (ctx_aug[:, :DH] / denom).astype(bf16)
        contrib = jnp.dot(ctx, wo, preferred_element_type=f32)
        sl = pl.ds(qb * S_LOC, S_LOC)

        @pl.when(h == 0)
        def _():
            out_ref[0, sl, :] = contrib

        @pl.when(h != 0)
        def _():
            out_ref[0, sl, :] = out_ref[0, sl, :] + contrib


def _attention_partial(x_full, wq, wk, wv, wo):
    return pl.pallas_call(
        _attn_body,
        grid=(B, H_LOC),
        out_shape=jax.ShapeDtypeStruct((B, S, D), jnp.float32),
        in_specs=[
            pl.BlockSpec((1, S, D), lambda b, h: (b, 0, 0)),
            pl.BlockSpec((D, DH), lambda b, h: (0, h)),
            pl.BlockSpec((D, DH), lambda b, h: (0, h)),
            pl.BlockSpec((D, DH), lambda b, h: (0, h)),
            pl.BlockSpec((DH, D), lambda b, h: (h, 0)),
        ],
        out_specs=pl.BlockSpec((1, S, D), lambda b, h: (b, 0, 0)),
        compiler_params=pltpu.CompilerParams(
            vmem_limit_bytes=64 * 1024 * 1024,
        ),
    )(x_full, wq, wk, wv, wo)


def _rs_body(p_ref, out_ref, rsbuf, sendbuf, send_sems, recv_sems):
    my = lax.axis_index("i")
    left = lax.rem(my + N_DEV - 1, N_DEV)
    right = lax.rem(my + 1, N_DEV)

    _neighbor_barrier(left, right)

    c0 = lax.rem(my + N_DEV - 1, N_DEV)
    sendbuf[...] = p_ref[:, pl.ds(c0 * S_LOC, S_LOC), :].astype(jnp.bfloat16)
    for s in range(N_DEV - 1):
        rdma = pltpu.make_async_remote_copy(
            src_ref=sendbuf,
            dst_ref=rsbuf.at[s],
            send_sem=send_sems.at[s],
            recv_sem=recv_sems.at[s],
            device_id=(right,),
            device_id_type=pl.DeviceIdType.MESH,
        )
        rdma.start()
        rdma.wait()
        c_recv = lax.rem(my + 2 * N_DEV - 2 - s, N_DEV)
        csl = pl.ds(c_recv * S_LOC, S_LOC)
        acc = rsbuf[s].astype(jnp.float32) + p_ref[:, csl, :]
        if s < N_DEV - 2:
            sendbuf[...] = acc.astype(jnp.bfloat16)
        else:
            out_ref[...] = acc


def _reduce_scatter(partial):
    return pl.pallas_call(
        _rs_body,
        out_shape=jax.ShapeDtypeStruct((B, S_LOC, D), jnp.float32),
        in_specs=[pl.BlockSpec(memory_space=pltpu.VMEM)],
        out_specs=pl.BlockSpec(memory_space=pltpu.VMEM),
        scratch_shapes=[
            pltpu.VMEM((N_DEV - 1, B, S_LOC, D), jnp.bfloat16),
            pltpu.VMEM((B, S_LOC, D), jnp.bfloat16),
            pltpu.SemaphoreType.DMA((N_DEV - 1,)),
            pltpu.SemaphoreType.DMA((N_DEV - 1,)),
        ],
        compiler_params=pltpu.CompilerParams(collective_id=1),
    )(partial)


def kernel(x, Wq, Wk, Wv, Wo):
    x_full = _all_gather(x)
    partial = _attention_partial(x_full, Wq, Wk, Wv, Wo)
    return _reduce_scatter(partial)


# device time: 283659 ns/iter; 1.5321x vs baseline; 1.5321x over previous
import jax
import jax.numpy as jnp
from jax import lax
from jax.experimental import pallas as pl
from jax.experimental.pallas import tpu as pltpu

N_DEV = 4
B = 2
S_LOC = 512
S = 2048
D = 1024
H_LOC = 8
DH = 128
S_BLK = 256
SCALE = 0.08838834764831843


def _neighbor_barrier(left, right):
    barrier = pltpu.get_barrier_semaphore()
    for nbr in (left, right):
        pl.semaphore_signal(
            barrier, inc=1, device_id=(nbr,),
            device_id_type=pl.DeviceIdType.MESH,
        )
    pl.semaphore_wait(barrier, 2)


def _rope_piece(t, offset):
    R = t.shape[0]
    pos = (lax.broadcasted_iota(jnp.int32, (R, DH), 0) + offset).astype(
        jnp.float32
    )
    d = lax.broadcasted_iota(jnp.int32, (R, DH), 1)
    half = (d // 2).astype(jnp.float32)
    inv = jnp.exp(-jnp.log(10000.0) * (half * (2.0 / DH)))
    ang = pos * inv
    l = jnp.concatenate([t[:, 1:], t[:, :1]], axis=1)
    r = jnp.concatenate([t[:, -1:], t[:, :-1]], axis=1)
    tr = jnp.where(d % 2 == 0, -l, r)
    return t * jnp.cos(ang) + tr * jnp.sin(ang)


def _fused_body(x_ref, wq_ref, wk_ref, wv_ref, wo_ref, out_ref,
                xg, kbuf, vbuf, rsbuf, sendbuf, chunkacc, qbuf,
                ag_send, ag_recv, rs_send, rs_recv):
    f32 = jnp.float32
    bf16 = jnp.bfloat16
    my = lax.axis_index("i")
    left = lax.rem(my + N_DEV - 1, N_DEV)
    right = lax.rem(my + 1, N_DEV)

    _neighbor_barrier(left, right)

    ones = jnp.ones((S, 8), bf16)

    def kv_for_chunk(c):
        rows = pl.ds(c * S_LOC, S_LOC)
        for b in range(B):
            xcb = xg[c, b]
            kf = jnp.dot(xcb, wk_ref[...], preferred_element_type=f32)
            for hh in range(H_LOC):
                piece = _rope_piece(kf[:, hh * DH:(hh + 1) * DH], c * S_LOC)
                kbuf[b, rows, hh * DH:(hh + 1) * DH] = piece.astype(bf16)
            vf = jnp.dot(xcb, wv_ref[...], preferred_element_type=f32)
            vbuf[b, rows, :] = vf.astype(bf16)

    xg[my] = x_ref[...]
    for h in range(N_DEV - 1):
        origin = lax.rem(my + N_DEV - h, N_DEV)
        rdma = pltpu.make_async_remote_copy(
            src_ref=xg.at[origin],
            dst_ref=xg.at[origin],
            send_sem=ag_send.at[h],
            recv_sem=ag_recv.at[h],
            device_id=(right,),
            device_id_type=pl.DeviceIdType.MESH,
        )
        rdma.start()
        kv_for_chunk(origin)
        rdma.wait()
    kv_for_chunk(lax.rem(my + 1, N_DEV))

    def chunk_contrib_b(c, b):
        qf = jnp.dot(xg[c, b], wq_ref[...], preferred_element_type=f32)
        for hh in range(H_LOC):
            piece = _rope_piece(qf[:, hh * DH:(hh + 1) * DH], c * S_LOC)
            qbuf[:, hh * DH:(hh + 1) * DH] = (piece * SCALE).astype(bf16)
        chunkacc[...] = jnp.zeros((S_LOC, D), f32)

        def hbody(h, _):
            hs = pl.ds(h * DH, DH)
            k_bh = kbuf[b, :, hs]
            v_aug = jnp.concatenate([vbuf[b, :, hs], ones], axis=1)
            wo_h = wo_ref[hs, :]
            for sb in range(S_LOC // S_BLK):
                rows = pl.ds(sb * S_BLK, S_BLK)
                qs = qbuf[rows, hs]
                s = lax.dot_general(
                    qs, k_bh, (((1,), (1,)), ((), ())),
                    preferred_element_type=f32,
                )
                p = jnp.exp(s).astype(bf16)
                ctx_aug = jnp.dot(p, v_aug, preferred_element_type=f32)
                denom = ctx_aug[:, DH:DH + 1]
                ctx = (ctx_aug[:, :DH] / denom).astype(bf16)
                contrib = jnp.dot(ctx, wo_h, preferred_element_type=f32)
                chunkacc[rows, :] = chunkacc[rows, :] + contrib
            return 0

        lax.fori_loop(0, H_LOC, hbody, 0)

    def rs_rdma(idx):
        return pltpu.make_async_remote_copy(
            src_ref=sendbuf,
            dst_ref=rsbuf.at[idx],
            send_sem=rs_send.at[idx],
            recv_sem=rs_recv.at[idx],
            device_id=(right,),
            device_id_type=pl.DeviceIdType.MESH,
        )

    c = lax.rem(my + N_DEV - 1, N_DEV)
    for b in range(B):
        chunk_contrib_b(c, b)
        sendbuf[b] = chunkacc[...].astype(bf16)
    rs0 = rs_rdma(0)
    rs0.start()

    c = lax.rem(my + N_DEV - 2, N_DEV)
    for b in range(B):
        chunk_contrib_b(c, b)
        sendbuf[b] = chunkacc[...].astype(bf16)
    rs0.wait()
    sendbuf[...] = (
        sendbuf[...].astype(f32) + rsbuf[0].astype(f32)
    ).astype(bf16)
    rs1 = rs_rdma(1)
    rs1.start()

    c = lax.rem(my + N_DEV - 3, N_DEV)
    for b in range(B):
        chunk_contrib_b(c, b)
        sendbuf[b] = chunkacc[...].astype(bf16)
    rs1.wait()
    sendbuf[...] = (
        sendbuf[...].astype(f32) + rsbuf[1].astype(f32)
    ).astype(bf16)
    rs2 = rs_rdma(2)
    rs2.start()

    for b in range(B):
        chunk_contrib_b(my, b)
        out_ref[b] = chunkacc[...]
    rs2.wait()
    out_ref[...] = out_ref[...] + rsbuf[2].astype(f32)


def kernel(x, Wq, Wk, Wv, Wo):
    bf16 = jnp.bfloat16
    return pl.pallas_call(
        _fused_body,
        out_shape=jax.ShapeDtypeStruct((B, S_LOC, D), jnp.float32),
        in_specs=[pl.BlockSpec(memory_space=pltpu.VMEM)] * 5,
        out_specs=pl.BlockSpec(memory_space=pltpu.VMEM),
        scratch_shapes=[
            pltpu.VMEM((N_DEV, B, S_LOC, D), bf16),
            pltpu.VMEM((B, S, D), bf16),
            pltpu.VMEM((B, S, D), bf16),
            pltpu.VMEM((N_DEV - 1, B, S_LOC, D), bf16),
            pltpu.VMEM((B, S_LOC, D), bf16),
            pltpu.VMEM((S_LOC, D), jnp.float32),
            pltpu.VMEM((S_LOC, D), bf16),
            pltpu.SemaphoreType.DMA((N_DEV - 1,)),
            pltpu.SemaphoreType.DMA((N_DEV - 1,)),
            pltpu.SemaphoreType.DMA((N_DEV - 1,)),
            pltpu.SemaphoreType.DMA((N_DEV - 1,)),
        ],
        compiler_params=pltpu.CompilerParams(
            collective_id=0,
            vmem_limit_bytes=64 * 1024 * 1024,
        ),
    )(
        x.astype(bf16),
        Wq.astype(bf16),
        Wk.astype(bf16),
        Wv.astype(bf16),
        Wo.astype(bf16),
    )


# device time: 245889 ns/iter; 1.7674x vs baseline; 1.1536x over previous
import jax
import jax.numpy as jnp
from jax import lax
from jax.experimental import pallas as pl
from jax.experimental.pallas import tpu as pltpu

N_DEV = 4
B = 2
S_LOC = 512
S = 2048
D = 1024
H_LOC = 8
DH = 128
S_BLK = 256
SCALE = 0.08838834764831843


def _neighbor_barrier(left, right):
    barrier = pltpu.get_barrier_semaphore()
    for nbr in (left, right):
        pl.semaphore_signal(
            barrier, inc=1, device_id=(nbr,),
            device_id_type=pl.DeviceIdType.MESH,
        )
    pl.semaphore_wait(barrier, 2)


def _rope_piece(t, offset):
    R = t.shape[0]
    pos = (lax.broadcasted_iota(jnp.int32, (R, DH), 0) + offset).astype(
        jnp.float32
    )
    d = lax.broadcasted_iota(jnp.int32, (R, DH), 1)
    half = (d // 2).astype(jnp.float32)
    inv = jnp.exp(-jnp.log(10000.0) * (half * (2.0 / DH)))
    ang = pos * inv
    l = jnp.concatenate([t[:, 1:], t[:, :1]], axis=1)
    r = jnp.concatenate([t[:, -1:], t[:, :-1]], axis=1)
    tr = jnp.where(d % 2 == 0, -l, r)
    return t * jnp.cos(ang) + tr * jnp.sin(ang)


def _fused_body(x_ref, wq_ref, wk_ref, wv_ref, wo_ref, out_ref,
                xg, kbuf, vbuf, rsbuf, sendbuf, qbuf, ctxbuf,
                ag_send, ag_recv, rs_send, rs_recv):
    f32 = jnp.float32
    bf16 = jnp.bfloat16
    my = lax.axis_index("i")
    left = lax.rem(my + N_DEV - 1, N_DEV)
    right = lax.rem(my + 1, N_DEV)

    _neighbor_barrier(left, right)

    def kv_for_chunk(c):
        rows = pl.ds(c * S_LOC, S_LOC)
        for b in range(B):
            xcb = xg[c, b]
            kf = jnp.dot(xcb, wk_ref[...], preferred_element_type=f32)
            for hh in range(H_LOC):
                piece = _rope_piece(kf[:, hh * DH:(hh + 1) * DH], c * S_LOC)
                kbuf[b, rows, hh * DH:(hh + 1) * DH] = piece.astype(bf16)
            vf = jnp.dot(xcb, wv_ref[...], preferred_element_type=f32)
            vbuf[b, rows, :] = vf.astype(bf16)

    xg[my] = x_ref[...]
    for h in range(N_DEV - 1):
        origin = lax.rem(my + N_DEV - h, N_DEV)
        rdma = pltpu.make_async_remote_copy(
            src_ref=xg.at[origin],
            dst_ref=xg.at[origin],
            send_sem=ag_send.at[h],
            recv_sem=ag_recv.at[h],
            device_id=(right,),
            device_id_type=pl.DeviceIdType.MESH,
        )
        rdma.start()
        kv_for_chunk(origin)
        rdma.wait()
    kv_for_chunk(lax.rem(my + 1, N_DEV))

    def chunk_contrib_b(c, b, write_out):
        qf = jnp.dot(xg[c, b], wq_ref[...], preferred_element_type=f32)
        for hh in range(H_LOC):
            piece = _rope_piece(qf[:, hh * DH:(hh + 1) * DH], c * S_LOC)
            qbuf[:, hh * DH:(hh + 1) * DH] = (piece * SCALE).astype(bf16)

        for sb in range(S_LOC // S_BLK):
            rows = pl.ds(sb * S_BLK, S_BLK)

            def hbody(h, _):
                hs = pl.ds(h * DH, DH)
                k_bh = kbuf[b, :, hs]
                v_bh = vbuf[b, :, hs]
                qs = qbuf[rows, hs]
                s = lax.dot_general(
                    qs, k_bh, (((1,), (1,)), ((), ())),
                    preferred_element_type=f32,
                )
                p32 = jnp.exp(s)
                denom = jnp.sum(p32, axis=1, keepdims=True)
                ctx_un = jnp.dot(
                    p32.astype(bf16), v_bh, preferred_element_type=f32
                )
                ctxbuf[:, hs] = (ctx_un / denom).astype(bf16)
                return 0

            lax.fori_loop(0, H_LOC, hbody, 0)
            contrib = jnp.dot(
                ctxbuf[...], wo_ref[...], preferred_element_type=f32
            )
            if write_out:
                out_ref[b, rows, :] = contrib
            else:
                sendbuf[b, rows, :] = contrib.astype(bf16)

    def rs_rdma(idx):
        return pltpu.make_async_remote_copy(
            src_ref=sendbuf,
            dst_ref=rsbuf.at[idx],
            send_sem=rs_send.at[idx],
            recv_sem=rs_recv.at[idx],
            device_id=(right,),
            device_id_type=pl.DeviceIdType.MESH,
        )

    c = lax.rem(my + N_DEV - 1, N_DEV)
    for b in range(B):
        chunk_contrib_b(c, b, False)
    rs0 = rs_rdma(0)
    rs0.start()

    c = lax.rem(my + N_DEV - 2, N_DEV)
    for b in range(B):
        chunk_contrib_b(c, b, False)
    rs0.wait()
    sendbuf[...] = (
        sendbuf[...].astype(f32) + rsbuf[0].astype(f32)
    ).astype(bf16)
    rs1 = rs_rdma(1)
    rs1.start()

    c = lax.rem(my + N_DEV - 3, N_DEV)
    for b in range(B):
        chunk_contrib_b(c, b, False)
    rs1.wait()
    sendbuf[...] = (
        sendbuf[...].astype(f32) + rsbuf[1].astype(f32)
    ).astype(bf16)
    rs2 = rs_rdma(2)
    rs2.start()

    for b in range(B):
        chunk_contrib_b(my, b, True)
    rs2.wait()
    out_ref[...] = out_ref[...] + rsbuf[2].astype(f32)


def kernel(x, Wq, Wk, Wv, Wo):
    bf16 = jnp.bfloat16
    return pl.pallas_call(
        _fused_body,
        out_shape=jax.ShapeDtypeStruct((B, S_LOC, D), jnp.float32),
        in_specs=[pl.BlockSpec(memory_space=pltpu.VMEM)] * 5,
        out_specs=pl.BlockSpec(memory_space=pltpu.VMEM),
        scratch_shapes=[
            pltpu.VMEM((N_DEV, B, S_LOC, D), bf16),
            pltpu.VMEM((B, S, D), bf16),
            pltpu.VMEM((B, S, D), bf16),
            pltpu.VMEM((N_DEV - 1, B, S_LOC, D), bf16),
            pltpu.VMEM((B, S_LOC, D), bf16),
            pltpu.VMEM((S_LOC, D), bf16),
            pltpu.VMEM((S_BLK, D), bf16),
            pltpu.SemaphoreType.DMA((N_DEV - 1,)),
            pltpu.SemaphoreType.DMA((N_DEV - 1,)),
            pltpu.SemaphoreType.DMA((N_DEV - 1,)),
            pltpu.SemaphoreType.DMA((N_DEV - 1,)),
        ],
        compiler_params=pltpu.CompilerParams(
            collective_id=0,
            vmem_limit_bytes=64 * 1024 * 1024,
        ),
    )(
        x.astype(bf16),
        Wq.astype(bf16),
        Wk.astype(bf16),
        Wv.astype(bf16),
        Wo.astype(bf16),
    )


# device time: 225233 ns/iter; 1.9295x vs baseline; 1.0917x over previous
import jax
import jax.numpy as jnp
from jax import lax
from jax.experimental import pallas as pl
from jax.experimental.pallas import tpu as pltpu

N_DEV = 4
B = 2
S_LOC = 512
S = 2048
D = 1024
H_LOC = 8
DH = 128
S_BLK = 512
SCALE = 0.08838834764831843


def _neighbor_barrier(left, right):
    barrier = pltpu.get_barrier_semaphore()
    for nbr in (left, right):
        pl.semaphore_signal(
            barrier, inc=1, device_id=(nbr,),
            device_id_type=pl.DeviceIdType.MESH,
        )
    pl.semaphore_wait(barrier, 2)


def _rope_piece(t, offset):
    R = t.shape[0]
    pos = (lax.broadcasted_iota(jnp.int32, (R, DH), 0) + offset).astype(
        jnp.float32
    )
    d = lax.broadcasted_iota(jnp.int32, (R, DH), 1)
    half = (d // 2).astype(jnp.float32)
    inv = jnp.exp(-jnp.log(10000.0) * (half * (2.0 / DH)))
    ang = pos * inv
    l = jnp.concatenate([t[:, 1:], t[:, :1]], axis=1)
    r = jnp.concatenate([t[:, -1:], t[:, :-1]], axis=1)
    tr = jnp.where(d % 2 == 0, -l, r)
    return t * jnp.cos(ang) + tr * jnp.sin(ang)


def _fused_body(x_ref, wq_ref, wk_ref, wv_ref, wo_ref, out_ref,
                xg, kbuf, vbuf, rsbuf, sendbuf, qbuf, ctxbuf,
                ag_send, ag_recv, rs_send, rs_recv):
    f32 = jnp.float32
    bf16 = jnp.bfloat16
    my = lax.axis_index("i")
    left = lax.rem(my + N_DEV - 1, N_DEV)
    right = lax.rem(my + 1, N_DEV)

    _neighbor_barrier(left, right)

    def kv_for_chunk(c):
        rows = pl.ds(c * S_LOC, S_LOC)
        for b in range(B):
            xcb = xg[c, b]
            kf = jnp.dot(xcb, wk_ref[...], preferred_element_type=f32)
            for hh in range(H_LOC):
                piece = _rope_piece(kf[:, hh * DH:(hh + 1) * DH], c * S_LOC)
                kbuf[b, rows, hh * DH:(hh + 1) * DH] = piece.astype(bf16)
            vf = jnp.dot(xcb, wv_ref[...], preferred_element_type=f32)
            vbuf[b, rows, :] = vf.astype(bf16)

    xg[my] = x_ref[...]
    for h in range(N_DEV - 1):
        origin = lax.rem(my + N_DEV - h, N_DEV)
        rdma = pltpu.make_async_remote_copy(
            src_ref=xg.at[origin],
            dst_ref=xg.at[origin],
            send_sem=ag_send.at[h],
            recv_sem=ag_recv.at[h],
            device_id=(right,),
            device_id_type=pl.DeviceIdType.MESH,
        )
        rdma.start()
        kv_for_chunk(origin)
        rdma.wait()
    kv_for_chunk(lax.rem(my + 1, N_DEV))

    def chunk_contrib_b(c, b, write_out):
        qf = jnp.dot(xg[c, b], wq_ref[...], preferred_element_type=f32)
        for hh in range(H_LOC):
            piece = _rope_piece(qf[:, hh * DH:(hh + 1) * DH], c * S_LOC)
            qbuf[:, hh * DH:(hh + 1) * DH] = (piece * SCALE).astype(bf16)

        for sb in range(S_LOC // S_BLK):
            rows = pl.ds(sb * S_BLK, S_BLK)

            def hbody(h, _):
                hs = pl.ds(h * DH, DH)
                k_bh = kbuf[b, :, hs]
                v_bh = vbuf[b, :, hs]
                qs = qbuf[rows, hs]
                s = lax.dot_general(
                    qs, k_bh, (((1,), (1,)), ((), ())),
                    preferred_element_type=f32,
                )
                p32 = jnp.exp(s)
                denom = jnp.sum(p32, axis=1, keepdims=True)
                ctx_un = jnp.dot(
                    p32.astype(bf16), v_bh, preferred_element_type=f32
                )
                ctxbuf[:, hs] = (ctx_un / denom).astype(bf16)
                return 0

            lax.fori_loop(0, H_LOC, hbody, 0)
            contrib = jnp.dot(
                ctxbuf[...], wo_ref[...], preferred_element_type=f32
            )
            if write_out:
                out_ref[b, rows, :] = contrib
            else:
                sendbuf[b, rows, :] = contrib.astype(bf16)

    def rs_rdma(idx):
        return pltpu.make_async_remote_copy(
            src_ref=sendbuf,
            dst_ref=rsbuf.at[idx],
            send_sem=rs_send.at[idx],
            recv_sem=rs_recv.at[idx],
            device_id=(right,),
            device_id_type=pl.DeviceIdType.MESH,
        )

    c = lax.rem(my + N_DEV - 1, N_DEV)
    for b in range(B):
        chunk_contrib_b(c, b, False)
    rs0 = rs_rdma(0)
    rs0.start()

    c = lax.rem(my + N_DEV - 2, N_DEV)
    for b in range(B):
        chunk_contrib_b(c, b, False)
    rs0.wait()
    sendbuf[...] = (
        sendbuf[...].astype(f32) + rsbuf[0].astype(f32)
    ).astype(bf16)
    rs1 = rs_rdma(1)
    rs1.start()

    c = lax.rem(my + N_DEV - 3, N_DEV)
    for b in range(B):
        chunk_contrib_b(c, b, False)
    rs1.wait()
    sendbuf[...] = (
        sendbuf[...].astype(f32) + rsbuf[1].astype(f32)
    ).astype(bf16)
    rs2 = rs_rdma(2)
    rs2.start()

    for b in range(B):
        chunk_contrib_b(my, b, True)
    rs2.wait()
    out_ref[...] = out_ref[...] + rsbuf[2].astype(f32)


def kernel(x, Wq, Wk, Wv, Wo):
    bf16 = jnp.bfloat16
    return pl.pallas_call(
        _fused_body,
        out_shape=jax.ShapeDtypeStruct((B, S_LOC, D), jnp.float32),
        in_specs=[pl.BlockSpec(memory_space=pltpu.VMEM)] * 5,
        out_specs=pl.BlockSpec(memory_space=pltpu.VMEM),
        scratch_shapes=[
            pltpu.VMEM((N_DEV, B, S_LOC, D), bf16),
            pltpu.VMEM((B, S, D), bf16),
            pltpu.VMEM((B, S, D), bf16),
            pltpu.VMEM((N_DEV - 1, B, S_LOC, D), bf16),
            pltpu.VMEM((B, S_LOC, D), bf16),
            pltpu.VMEM((S_LOC, D), bf16),
            pltpu.VMEM((S_BLK, D), bf16),
            pltpu.SemaphoreType.DMA((N_DEV - 1,)),
            pltpu.SemaphoreType.DMA((N_DEV - 1,)),
            pltpu.SemaphoreType.DMA((N_DEV - 1,)),
            pltpu.SemaphoreType.DMA((N_DEV - 1,)),
        ],
        compiler_params=pltpu.CompilerParams(
            collective_id=0,
            vmem_limit_bytes=64 * 1024 * 1024,
        ),
    )(
        x.astype(bf16),
        Wq.astype(bf16),
        Wk.astype(bf16),
        Wv.astype(bf16),
        Wo.astype(bf16),
    )


# device time: 224538 ns/iter; 1.9355x vs baseline; 1.0031x over previous
import jax
import jax.numpy as jnp
from jax import lax
from jax.experimental import pallas as pl
from jax.experimental.pallas import tpu as pltpu

N_DEV = 4
B = 2
S_LOC = 512
S = 2048
D = 1024
H_LOC = 8
DH = 128
S_BLK = 512
SCALE = 0.08838834764831843


def _neighbor_barrier(left, right):
    barrier = pltpu.get_barrier_semaphore()
    for nbr in (left, right):
        pl.semaphore_signal(
            barrier, inc=1, device_id=(nbr,),
            device_id_type=pl.DeviceIdType.MESH,
        )
    pl.semaphore_wait(barrier, 2)


def _rope_piece(t, offset):
    R = t.shape[0]
    pos = (lax.broadcasted_iota(jnp.int32, (R, DH), 0) + offset).astype(
        jnp.float32
    )
    d = lax.broadcasted_iota(jnp.int32, (R, DH), 1)
    half = (d // 2).astype(jnp.float32)
    inv = jnp.exp(-jnp.log(10000.0) * (half * (2.0 / DH)))
    ang = pos * inv
    l = jnp.concatenate([t[:, 1:], t[:, :1]], axis=1)
    r = jnp.concatenate([t[:, -1:], t[:, :-1]], axis=1)
    tr = jnp.where(d % 2 == 0, -l, r)
    return t * jnp.cos(ang) + tr * jnp.sin(ang)


def _fused_body(x_ref, wq_ref, wk_ref, wv_ref, wo_ref, out_ref,
                xg, kbuf, vbuf, qgbuf, rsbuf, sendbuf, ctxbuf,
                ag_send, ag_recv, rs_send, rs_recv):
    f32 = jnp.float32
    bf16 = jnp.bfloat16
    my = lax.axis_index("i")
    left = lax.rem(my + N_DEV - 1, N_DEV)
    right = lax.rem(my + 1, N_DEV)

    _neighbor_barrier(left, right)

    def kv_for_chunk(c):
        rows = pl.ds(c * S_LOC, S_LOC)
        for b in range(B):
            xcb = xg[c, b]
            kf = jnp.dot(xcb, wk_ref[...], preferred_element_type=f32)
            for hh in range(H_LOC):
                piece = _rope_piece(kf[:, hh * DH:(hh + 1) * DH], c * S_LOC)
                kbuf[b, rows, hh * DH:(hh + 1) * DH] = piece.astype(bf16)
            vf = jnp.dot(xcb, wv_ref[...], preferred_element_type=f32)
            vbuf[b, rows, :] = vf.astype(bf16)
            qf = jnp.dot(xcb, wq_ref[...], preferred_element_type=f32)
            for hh in range(H_LOC):
                piece = _rope_piece(qf[:, hh * DH:(hh + 1) * DH], c * S_LOC)
                qgbuf[b, rows, hh * DH:(hh + 1) * DH] = (
                    piece * SCALE
                ).astype(bf16)

    xg[my] = x_ref[...]
    for h in range(N_DEV - 1):
        origin = lax.rem(my + N_DEV - h, N_DEV)
        rdma = pltpu.make_async_remote_copy(
            src_ref=xg.at[origin],
            dst_ref=xg.at[origin],
            send_sem=ag_send.at[h],
            recv_sem=ag_recv.at[h],
            device_id=(right,),
            device_id_type=pl.DeviceIdType.MESH,
        )
        rdma.start()
        kv_for_chunk(origin)
        rdma.wait()
    kv_for_chunk(lax.rem(my + 1, N_DEV))

    def chunk_contrib_b(c, b, write_out):
        for sb in range(S_LOC // S_BLK):
            rows = pl.ds(sb * S_BLK, S_BLK)

            def hbody(h, _):
                hs = pl.ds(h * DH, DH)
                k_bh = kbuf[b, :, hs]
                v_bh = vbuf[b, :, hs]
                qs = qgbuf[b, pl.ds(c * S_LOC + sb * S_BLK, S_BLK), hs]
                s = lax.dot_general(
                    qs, k_bh, (((1,), (1,)), ((), ())),
                    preferred_element_type=f32,
                )
                p32 = jnp.exp(s)
                denom = jnp.sum(p32, axis=1, keepdims=True)
                ctx_un = jnp.dot(
                    p32.astype(bf16), v_bh, preferred_element_type=f32
                )
                ctxbuf[:, hs] = (ctx_un / denom).astype(bf16)
                return 0

            lax.fori_loop(0, H_LOC, hbody, 0)
            contrib = jnp.dot(
                ctxbuf[...], wo_ref[...], preferred_element_type=f32
            )
            if write_out:
                out_ref[b, rows, :] = contrib
            else:
                sendbuf[b, rows, :] = contrib.astype(bf16)

    def rs_rdma(idx):
        return pltpu.make_async_remote_copy(
            src_ref=sendbuf,
            dst_ref=rsbuf.at[idx],
            send_sem=rs_send.at[idx],
            recv_sem=rs_recv.at[idx],
            device_id=(right,),
            device_id_type=pl.DeviceIdType.MESH,
        )

    c = lax.rem(my + N_DEV - 1, N_DEV)
    for b in range(B):
        chunk_contrib_b(c, b, False)
    rs0 = rs_rdma(0)
    rs0.start()

    c = lax.rem(my + N_DEV - 2, N_DEV)
    for b in range(B):
        chunk_contrib_b(c, b, False)
    rs0.wait()
    sendbuf[...] = (
        sendbuf[...].astype(f32) + rsbuf[0].astype(f32)
    ).astype(bf16)
    rs1 = rs_rdma(1)
    rs1.start()

    c = lax.rem(my + N_DEV - 3, N_DEV)
    for b in range(B):
        chunk_contrib_b(c, b, False)
    rs1.wait()
    sendbuf[...] = (
        sendbuf[...].astype(f32) + rsbuf[1].astype(f32)
    ).astype(bf16)
    rs2 = rs_rdma(2)
    rs2.start()

    for b in range(B):
        chunk_contrib_b(my, b, True)
    rs2.wait()
    out_ref[...] = out_ref[...] + rsbuf[2].astype(f32)


def kernel(x, Wq, Wk, Wv, Wo):
    bf16 = jnp.bfloat16
    return pl.pallas_call(
        _fused_body,
        out_shape=jax.ShapeDtypeStruct((B, S_LOC, D), jnp.float32),
        in_specs=[pl.BlockSpec(memory_space=pltpu.VMEM)] * 5,
        out_specs=pl.BlockSpec(memory_space=pltpu.VMEM),
        scratch_shapes=[
            pltpu.VMEM((N_DEV, B, S_LOC, D), bf16),
            pltpu.VMEM((B, S, D), bf16),
            pltpu.VMEM((B, S, D), bf16),
            pltpu.VMEM((B, S, D), bf16),
            pltpu.VMEM((N_DEV - 1, B, S_LOC, D), bf16),
            pltpu.VMEM((B, S_LOC, D), bf16),
            pltpu.VMEM((S_BLK, D), bf16),
            pltpu.SemaphoreType.DMA((N_DEV - 1,)),
            pltpu.SemaphoreType.DMA((N_DEV - 1,)),
            pltpu.SemaphoreType.DMA((N_DEV - 1,)),
            pltpu.SemaphoreType.DMA((N_DEV - 1,)),
        ],
        compiler_params=pltpu.CompilerParams(
            collective_id=0,
            vmem_limit_bytes=64 * 1024 * 1024,
        ),
    )(
        x.astype(bf16),
        Wq.astype(bf16),
        Wk.astype(bf16),
        Wv.astype(bf16),
        Wo.astype(bf16),
    )
